# baseline (speedup 1.0000x reference)
import functools

import jax
import jax.numpy as jnp
from jax import lax
from jax.experimental import pallas as pl
from jax.experimental.pallas import tpu as pltpu
from jax.experimental.pallas import tpu_sc as plsc

BATCH = 4096
HIST = 200
EMBED_DIM = 64
ROWS = BATCH * HIST
CHUNK = 128
NCHUNK = ROWS // CHUNK
NWORK = 32
PER_W = NCHUNK // NWORK

_mesh = plsc.VectorSubcoreMesh(
    core_axis_name="c", subcore_axis_name="s", num_cores=2, num_subcores=16
)


@functools.partial(
    pl.kernel,
    out_type=jax.ShapeDtypeStruct((NCHUNK, CHUNK, EMBED_DIM), jnp.float32),
    mesh=_mesh,
    scratch_types=[
        pltpu.VMEM((CHUNK,), jnp.int32),
        pltpu.VMEM((CHUNK, EMBED_DIM), jnp.float32),
        pltpu.SemaphoreType.DMA,
    ],
    compiler_params=pltpu.CompilerParams(use_tc_tiling_on_sc=False),
)
def _embed_add(x_hbm, idx_hbm, table_hbm, out_hbm, idx_v, x_v, sem):
    wid = lax.axis_index("s") * 2 + lax.axis_index("c")

    def body(i, _):
        g = wid * PER_W + i
        pltpu.sync_copy(idx_hbm.at[g], idx_v)
        pltpu.sync_copy(x_hbm.at[g], x_v)
        pltpu.async_copy(table_hbm.at[idx_v], x_v, sem, add=True).wait()
        pltpu.sync_copy(x_v, out_hbm.at[g])
        return ()

    lax.fori_loop(0, PER_W, body, ())


def kernel(x, in_chan_matrix, embed_weight):
    xf = x.reshape(NCHUNK, CHUNK, EMBED_DIM)
    idx = in_chan_matrix.astype(jnp.int32).reshape(NCHUNK, CHUNK)
    out = _embed_add(xf, idx, embed_weight)
    return out.reshape(BATCH, HIST, EMBED_DIM)


# speedup vs baseline: 1.2057x; 1.2057x over previous
import functools

import jax
import jax.numpy as jnp
from jax import lax
from jax.experimental import pallas as pl
from jax.experimental.pallas import tpu as pltpu
from jax.experimental.pallas import tpu_sc as plsc

BATCH = 4096
HIST = 200
EMBED_DIM = 64
ROWS = BATCH * HIST
GROW = 128
NGROW = ROWS // GROW
NWORK = 32
PER_W = NGROW // NWORK
GPC = 4
NCHUNK = PER_W // GPC
NBUF = 3

_mesh = plsc.VectorSubcoreMesh(
    core_axis_name="c", subcore_axis_name="s", num_cores=2, num_subcores=16
)


@functools.partial(
    pl.kernel,
    out_type=jax.ShapeDtypeStruct((NGROW, GROW, EMBED_DIM), jnp.float32),
    mesh=_mesh,
    scratch_types=[
        pltpu.VMEM((PER_W, GROW), jnp.int32),
        pltpu.VMEM((NBUF, GPC, GROW, EMBED_DIM), jnp.float32),
        pltpu.SemaphoreType.DMA,
        pltpu.SemaphoreType.DMA,
        pltpu.SemaphoreType.DMA,
    ],
    compiler_params=pltpu.CompilerParams(use_tc_tiling_on_sc=False),
)
def _embed_add(x_hbm, idx_hbm, table_hbm, out_hbm, idx_v, x_v, sem_l, sem_g, sem_s):
    wid = lax.axis_index("s") * 2 + lax.axis_index("c")
    base = wid * PER_W

    pltpu.sync_copy(idx_hbm.at[wid], idx_v)

    pltpu.async_copy(x_hbm.at[pl.ds(base, GPC)], x_v.at[0], sem_l)
    pltpu.async_copy(x_hbm.at[pl.ds(base + GPC, GPC)], x_v.at[1], sem_l)

    def body(c, _):
        b = lax.rem(c, NBUF)
        g0 = base + c * GPC

        @pl.when(c + 2 < NCHUNK)
        def _load_ahead():
            bn = lax.rem(c + 2, NBUF)

            @pl.when(c >= 1)
            def _drain_store():
                pltpu.make_async_copy(
                    x_v.at[bn], out_hbm.at[pl.ds(g0 - GPC, GPC)], sem_s
                ).wait()

            pltpu.async_copy(
                x_hbm.at[pl.ds(g0 + 2 * GPC, GPC)], x_v.at[bn], sem_l
            )

        pltpu.make_async_copy(
            x_hbm.at[pl.ds(g0, GPC)], x_v.at[b], sem_l
        ).wait()

        gathers = [
            pltpu.async_copy(
                table_hbm.at[idx_v.at[c * GPC + k]], x_v.at[b, k], sem_g, add=True
            )
            for k in range(GPC)
        ]
        for d in gathers:
            d.wait()

        pltpu.async_copy(x_v.at[b], out_hbm.at[pl.ds(g0, GPC)], sem_s)
        return ()

    lax.fori_loop(0, NCHUNK, body, ())

    for c in range(NCHUNK - NBUF, NCHUNK):
        pltpu.make_async_copy(
            x_v.at[c % NBUF], out_hbm.at[pl.ds(base + c * GPC, GPC)], sem_s
        ).wait()


def kernel(x, in_chan_matrix, embed_weight):
    xf = x.reshape(NGROW, GROW, EMBED_DIM)
    idx = in_chan_matrix.astype(jnp.int32).reshape(NWORK, PER_W, GROW)
    out = _embed_add(xf, idx, embed_weight)
    return out.reshape(BATCH, HIST, EMBED_DIM)
